# baseline (speedup 1.0000x reference)
import jax
import jax.numpy as jnp
from jax import lax
from jax.experimental import pallas as pl
from jax.experimental.pallas import tpu as pltpu
from jax.experimental.pallas import tpu_sc as plsc

PRE_SEQ_LEN = 128
HIDDEN_SIZE = 2048
NUM_LAYERS = 24
EMBED_DIM = 2 * NUM_LAYERS * HIDDEN_SIZE
BATCH = 8
PREFIX_LEN = 128

NB = BATCH * PREFIX_LEN
NSEQ = 2
ROWS = NB // NSEQ
LAG = 16


def _body(gidx_hbm, table_hbm, out_hbm, gidx_s, sem):
    cid = lax.axis_index("c")
    pltpu.sync_copy(gidx_hbm, gidx_s)
    base = cid * ROWS

    @pl.loop(0, ROWS)
    def _issue(j):
        idx = gidx_s[base + j]
        pltpu.async_copy(table_hbm.at[idx], out_hbm.at[base + j], sem)

        @pl.when(j >= LAG)
        def _():
            pltpu.make_async_copy(table_hbm.at[0], out_hbm.at[0], sem).wait()

    @pl.loop(0, LAG)
    def _drain(j):
        pltpu.make_async_copy(table_hbm.at[0], out_hbm.at[0], sem).wait()


@jax.jit
def _run(gidx, table):
    mesh = plsc.ScalarSubcoreMesh(axis_name="c", num_cores=NSEQ)
    f = pl.kernel(
        _body,
        out_type=jax.ShapeDtypeStruct((NB, EMBED_DIM), jnp.float32),
        mesh=mesh,
        scratch_types=[
            pltpu.SMEM((NB,), jnp.int32),
            pltpu.SemaphoreType.DMA,
        ],
    )
    return f(gidx, table)


def kernel(prefix, table):
    gidx = prefix.reshape(NB).astype(jnp.int32)
    out2 = _run(gidx, table)
    return out2.reshape(BATCH, PREFIX_LEN, EMBED_DIM)


# speedup vs baseline: 40.9935x; 40.9935x over previous
import jax
import jax.numpy as jnp
from jax import lax
from jax.experimental import pallas as pl
from jax.experimental.pallas import tpu as pltpu
from jax.experimental.pallas import tpu_sc as plsc

PRE_SEQ_LEN = 128
HIDDEN_SIZE = 2048
NUM_LAYERS = 24
EMBED_DIM = 2 * NUM_LAYERS * HIDDEN_SIZE
BATCH = 8
PREFIX_LEN = 128

NB = BATCH * PREFIX_LEN
NW = 32
ROWS = NB // NW

RPS = 1
NBUF = 2
BUF_WORDS = 49152
CD = BUF_WORDS // RPS
NCH = EMBED_DIM // CD
NSTREAM = (ROWS // RPS) * NCH


def _body(gidx_hbm, table_hbm, out_hbm, gidx_v, bufs, sgs, sss):
    wid = lax.axis_index("s") * 2 + lax.axis_index("c")
    pltpu.sync_copy(gidx_hbm.at[wid], gidx_v)

    def g_start(s, p):
        col = (s % NCH) * CD
        pltpu.async_copy(table_hbm.at[gidx_v.at[s], pl.ds(col, CD)],
                         bufs[p], sgs[p])

    def g_wait(s, p):
        col = (s % NCH) * CD
        pltpu.make_async_copy(table_hbm.at[gidx_v.at[s], pl.ds(col, CD)],
                              bufs[p], sgs[p]).wait()

    def _dst(s):
        row = wid * ROWS + (s // NCH) * RPS
        col = (s % NCH) * CD
        return out_hbm.at[pl.ds(row, RPS), pl.ds(col, CD)]

    def s_start(s, p):
        pltpu.async_copy(bufs[p], _dst(s), sss[p])

    def s_wait(s, p):
        pltpu.make_async_copy(bufs[p], _dst(s), sss[p]).wait()

    DEPTH = NBUF // 2
    for s in range(DEPTH):
        g_start(s, s % NBUF)

    @pl.loop(0, NSTREAM // NBUF)
    def _iter(g):
        for p in range(NBUF):
            s = NBUF * g + p
            g_wait(s, p)

            @pl.when(s >= DEPTH)
            def _():
                s_wait(s - DEPTH, (p - DEPTH) % NBUF)

            @pl.when(s + DEPTH < NSTREAM)
            def _():
                g_start(s + DEPTH, (p + DEPTH) % NBUF)

            s_start(s, p)

    for s in range(NSTREAM - DEPTH, NSTREAM):
        s_wait(s, s % NBUF)


@jax.jit
def _run(gidx, table):
    mesh = plsc.VectorSubcoreMesh(core_axis_name="c", subcore_axis_name="s")
    f = pl.kernel(
        _body,
        out_type=jax.ShapeDtypeStruct((NB, EMBED_DIM), jnp.float32),
        mesh=mesh,
        scratch_types=[
            pltpu.VMEM((NSTREAM, RPS), jnp.int32),
            [pltpu.VMEM((RPS, CD), jnp.float32)] * NBUF,
            [pltpu.SemaphoreType.DMA] * NBUF,
            [pltpu.SemaphoreType.DMA] * NBUF,
        ],
    )
    return f(gidx, table)


def kernel(prefix, table):
    pf = prefix.reshape(NW, ROWS // RPS, 1, RPS).astype(jnp.int32)
    gidx = jnp.broadcast_to(pf, (NW, ROWS // RPS, NCH, RPS)).reshape(
        NW, NSTREAM, RPS)
    out2 = _run(gidx, table)
    return out2.reshape(BATCH, PREFIX_LEN, EMBED_DIM)
